# baseline (speedup 1.0000x reference)
import functools

import jax
import jax.numpy as jnp
from jax.experimental import pallas as pl
from jax.experimental.pallas import tpu as pltpu


def _sepconv_nchw_kernel(x_ref, a_ref, b_ref, o_ref, xp0, xpm, xpp, *,
                         H, W, C_in):
    HW = H * W
    PAD = W

    C_out = o_ref.shape[1]
    xb = x_ref[0].reshape(C_in, HW).astype(jnp.bfloat16)
    col = jax.lax.broadcasted_iota(jnp.int32, (C_in, HW), 1) % W
    zero = jnp.zeros_like(xb)
    xb_m = jnp.where(col != W - 1, xb, zero)
    xb_p = jnp.where(col != 0, xb, zero)

    z_pad = jnp.zeros((C_in, PAD), jnp.bfloat16)
    z_pad1 = jnp.zeros((C_in, PAD + 1), jnp.bfloat16)
    z_padm1 = jnp.zeros((C_in, PAD - 1), jnp.bfloat16)

    xp0[:, :PAD] = z_pad
    xp0[:, PAD + HW:] = z_pad
    xp0[:, PAD:PAD + HW] = xb

    xpm[:, :PAD + 1] = z_pad1
    xpm[:, PAD + 1 + HW:] = z_padm1
    xpm[:, PAD + 1:PAD + 1 + HW] = xb_m

    xpp[:, :PAD - 1] = z_padm1
    xpp[:, PAD - 1 + HW:] = z_pad1
    xpp[:, PAD - 1:PAD - 1 + HW] = xb_p

    rhs = jnp.concatenate([
        xpm[:, 0:HW], xpm[:, PAD:PAD + HW], xpm[:, 2 * PAD:2 * PAD + HW],
        xp0[:, 0:HW], xb, xp0[:, 2 * PAD:2 * PAD + HW],
        xpp[:, 0:HW], xpp[:, PAD:PAD + HW], xpp[:, 2 * PAD:2 * PAD + HW],
    ], axis=0)

    acc = jnp.dot(a_ref[...], rhs, preferred_element_type=jnp.float32)
    acc = acc + b_ref[:, :1]
    o_ref[0] = acc.reshape(C_out, H, W).astype(o_ref.dtype)


def kernel(x_nchw, dw_weight, bn_gamma, bn_beta, bn_mean, bn_var, pw_weight):
    N, C_in, H, W = x_nchw.shape
    C_out = pw_weight.shape[0]
    HW = H * W
    f32 = jnp.float32

    scale = bn_gamma.astype(f32) * jax.lax.rsqrt(bn_var.astype(f32) + 1e-5)
    dwf = dw_weight[:, 0, :, :].astype(f32) * scale[:, None, None]
    pwf = pw_weight[:, :, 0, 0].astype(f32)
    e = jnp.transpose(dwf, (2, 1, 0))
    a4 = e[:, :, None, :] * pwf[None, None, :, :]
    lhs = jnp.transpose(a4, (2, 0, 1, 3)).reshape(C_out, 9 * C_in)
    lhs = lhs.astype(jnp.bfloat16)
    bias = pwf @ (bn_beta.astype(f32) - bn_mean.astype(f32) * scale)
    bias = jnp.broadcast_to(bias[:, None], (C_out, 128))

    body = functools.partial(_sepconv_nchw_kernel, H=H, W=W, C_in=C_in)
    out = pl.pallas_call(
        body,
        out_shape=jax.ShapeDtypeStruct((N, C_out, H, W), x_nchw.dtype),
        grid=(N,),
        in_specs=[
            pl.BlockSpec((1, C_in, H, W), lambda n: (n, 0, 0, 0)),
            pl.BlockSpec((C_out, 9 * C_in), lambda n: (0, 0)),
            pl.BlockSpec((C_out, 128), lambda n: (0, 0)),
        ],
        out_specs=pl.BlockSpec((1, C_out, H, W), lambda n: (n, 0, 0, 0)),
        scratch_shapes=[
            pltpu.VMEM((C_in, HW + 2 * W), jnp.bfloat16),
            pltpu.VMEM((C_in, HW + 2 * W), jnp.bfloat16),
            pltpu.VMEM((C_in, HW + 2 * W), jnp.bfloat16),
        ],
        compiler_params=pltpu.CompilerParams(
            dimension_semantics=("parallel",),
            vmem_limit_bytes=64 * 1024 * 1024,
        ),
    )(x_nchw, lhs, bias)
    return out


# speedup vs baseline: 3.3710x; 3.3710x over previous
import functools

import jax
import jax.numpy as jnp
from jax.experimental import pallas as pl
from jax.experimental.pallas import tpu as pltpu


def _sepconv_nhwc_kernel(x_ref, a_ref, b_ref, o_ref, xp0, xpm, xpp, *,
                         H, W, C_in):
    HW = H * W
    PAD = W

    xb = x_ref[0].reshape(HW, C_in)
    row = jax.lax.broadcasted_iota(jnp.int32, (HW, C_in), 0) % W
    zero = jnp.zeros_like(xb)
    xb_m = jnp.where(row != W - 1, xb, zero)
    xb_p = jnp.where(row != 0, xb, zero)

    z_pad = jnp.zeros((PAD, C_in), jnp.bfloat16)
    z_pad1 = jnp.zeros((PAD + 1, C_in), jnp.bfloat16)
    z_padm1 = jnp.zeros((PAD - 1, C_in), jnp.bfloat16)

    xp0[:PAD] = z_pad
    xp0[PAD + HW:] = z_pad
    xp0[PAD:PAD + HW] = xb

    xpm[:PAD + 1] = z_pad1
    xpm[PAD + 1 + HW:] = z_padm1
    xpm[PAD + 1:PAD + 1 + HW] = xb_m

    xpp[:PAD - 1] = z_padm1
    xpp[PAD - 1 + HW:] = z_pad1
    xpp[PAD - 1:PAD - 1 + HW] = xb_p

    xs = jnp.concatenate([
        xpm[0:HW], xpm[PAD:PAD + HW], xpm[2 * PAD:2 * PAD + HW],
        xp0[0:HW], xb, xp0[2 * PAD:2 * PAD + HW],
        xpp[0:HW], xpp[PAD:PAD + HW], xpp[2 * PAD:2 * PAD + HW],
    ], axis=1)

    acc = jnp.dot(xs, a_ref[...], preferred_element_type=jnp.float32)
    acc = acc + b_ref[...]
    o_ref[0] = acc.reshape(H, W, -1).astype(o_ref.dtype)


def kernel(x_nchw, dw_weight, bn_gamma, bn_beta, bn_mean, bn_var, pw_weight):
    N, C_in, H, W = x_nchw.shape
    C_out = pw_weight.shape[0]
    HW = H * W
    f32 = jnp.float32

    scale = bn_gamma.astype(f32) * jax.lax.rsqrt(bn_var.astype(f32) + 1e-5)
    dwf = dw_weight[:, 0, :, :].astype(f32) * scale[:, None, None]
    pwf = pw_weight[:, :, 0, 0].astype(f32)
    e = jnp.transpose(dwf, (2, 1, 0))
    a4 = e[:, :, :, None] * jnp.transpose(pwf)[None, None, :, :]
    a_stack = a4.reshape(9 * C_in, C_out).astype(jnp.bfloat16)
    bias = (pwf @ (bn_beta.astype(f32) - bn_mean.astype(f32) * scale))[None, :]

    xt = jnp.transpose(x_nchw, (0, 2, 3, 1)).astype(jnp.bfloat16)

    body = functools.partial(_sepconv_nhwc_kernel, H=H, W=W, C_in=C_in)
    out = pl.pallas_call(
        body,
        out_shape=jax.ShapeDtypeStruct((N, H, W, C_out), x_nchw.dtype),
        grid=(N,),
        in_specs=[
            pl.BlockSpec((1, H, W, C_in), lambda n: (n, 0, 0, 0)),
            pl.BlockSpec((9 * C_in, C_out), lambda n: (0, 0)),
            pl.BlockSpec((1, C_out), lambda n: (0, 0)),
        ],
        out_specs=pl.BlockSpec((1, H, W, C_out), lambda n: (n, 0, 0, 0)),
        scratch_shapes=[
            pltpu.VMEM((HW + 2 * W, C_in), jnp.bfloat16),
            pltpu.VMEM((HW + 2 * W, C_in), jnp.bfloat16),
            pltpu.VMEM((HW + 2 * W, C_in), jnp.bfloat16),
        ],
        compiler_params=pltpu.CompilerParams(
            dimension_semantics=("parallel",),
            vmem_limit_bytes=64 * 1024 * 1024,
        ),
    )(xt, a_stack, bias)
    return jnp.transpose(out, (0, 3, 1, 2))
